# baseline (speedup 1.0000x reference)
import jax
import jax.numpy as jnp
from jax import lax
from jax.experimental import pallas as pl
from jax.experimental.pallas import tpu as pltpu
from jax.experimental.pallas import tpu_sc as plsc

NKNOTS = 4096
NINT = NKNOTS - 1
INV_H = NINT / 2.0
N_TOTAL = 8388608
NC, NS, L = 2, 16, 16
NW = NC * NS
CHUNK = N_TOTAL // NW
BLK = 16384
NBLK = CHUNK // BLK
VPB = BLK // L
MASK_HI = jnp.int32(-65536)
RND = jnp.int32(0x8000)


def _spline_body(x_hbm, knots_hbm, coeffs_hbm, out_hbm,
                 y_v, kn_v, h_v, dl_v, d_v, b_v, ce_v,
                 xin0, xin1, outb0, outb1, si0, si1, so0, so1):
    wid = lax.axis_index("s") * NC + lax.axis_index("c")
    base = wid * CHUNK
    lanes = lax.iota(jnp.int32, L)

    pltpu.sync_copy(coeffs_hbm, y_v)
    pltpu.sync_copy(knots_hbm, kn_v)

    @plsc.parallel_loop(0, NKNOTS // L, unroll=4)
    def pass1(k):
        i0 = k * L
        idx = lanes + i0
        idxp = jnp.minimum(idx + 1, NKNOTS - 1)
        y0 = y_v[pl.ds(i0, L)]
        y1 = plsc.load_gather(y_v, [idxp])
        k0 = kn_v[pl.ds(i0, L)]
        k1 = plsc.load_gather(kn_v, [idxp])
        hh = (k1 - k0) + 1e-12
        h_v[pl.ds(i0, L)] = hh
        dl_v[pl.ds(i0, L)] = (y1 - y0) / hh

    @plsc.parallel_loop(0, NKNOTS // L, unroll=4)
    def pass2(k):
        i0 = k * L
        idx = lanes + i0
        idxm = jnp.maximum(idx - 1, 0)
        dln = dl_v[pl.ds(i0, L)]
        dlp = plsc.load_gather(dl_v, [idxm])
        hn = h_v[pl.ds(i0, L)]
        hp = plsc.load_gather(h_v, [idxm])
        same = (dlp * dln) > 0.0
        w1 = 2.0 * hn + hp
        w2 = hn + 2.0 * hp
        den = w1 / (dlp + 1e-12) + w2 / (dln + 1e-12) + 1e-12
        d = jnp.where(same, (w1 + w2) / den, 0.0)
        d = jnp.where(idx == 0, dln, d)
        d = jnp.where(idx == NKNOTS - 1, dlp, d)
        d_v[pl.ds(i0, L)] = d

    @plsc.parallel_loop(0, NKNOTS // L, unroll=4)
    def pass3(k):
        i0 = k * L
        idx = lanes + i0
        idxp = jnp.minimum(idx + 1, NKNOTS - 1)
        y0 = y_v[pl.ds(i0, L)]
        y1 = plsc.load_gather(y_v, [idxp])
        d0 = d_v[pl.ds(i0, L)]
        d1 = plsc.load_gather(d_v, [idxp])
        hh = h_v[pl.ds(i0, L)]
        b = hh * d0
        hd1 = hh * d1
        dy = y1 - y0
        c = 3.0 * dy - 2.0 * b - hd1
        e = -2.0 * dy + b + hd1
        cbits = plsc.bitcast(c, jnp.int32)
        ebits = plsc.bitcast(e, jnp.int32)
        w = ((cbits + RND) & MASK_HI) | lax.shift_right_logical(
            ebits + RND, 16)
        b_v[pl.ds(i0, L)] = b
        ce_v[pl.ds(i0, L)] = w

    c4095 = jnp.full((L,), NKNOTS - 1, jnp.int32)
    c4094 = jnp.full((L,), NKNOTS - 2, jnp.int32)
    yl = plsc.load_gather(y_v, [c4095])
    shi = plsc.load_gather(h_v, [c4094]) * plsc.load_gather(d_v, [c4095])

    def compute_block(src_ref, dst_ref):
        @plsc.parallel_loop(0, VPB, unroll=16)
        def inner(i):
            xo = i * L
            xv = src_ref[pl.ds(xo, L)]
            ff = xv * INV_H + INV_H
            ffc = jnp.minimum(jnp.maximum(ff, 0.0), float(NINT - 1))
            fi = ffc.astype(jnp.int32)
            t = ff - fi.astype(jnp.float32)
            a = plsc.load_gather(y_v, [fi])
            b = plsc.load_gather(b_v, [fi])
            w = plsc.load_gather(ce_v, [fi])
            c = plsc.bitcast(w & MASK_HI, jnp.float32)
            e = plsc.bitcast(lax.shift_left(w, 16), jnp.float32)
            tcl = jnp.minimum(jnp.maximum(t, 0.0), 1.0)
            dt = t - tcl
            cub = a + tcl * (b + tcl * (c + tcl * e))
            bse = jnp.where(dt > 0.0, yl, cub)
            slp = jnp.where(dt < 0.0, b, shi)
            dst_ref[pl.ds(xo, L)] = bse + slp * dt

    xin = (xin0, xin1)
    outb = (outb0, outb1)
    si = (si0, si1)
    so = (so0, so1)

    for j in range(2):
        pltpu.async_copy(x_hbm.at[pl.ds(base + j * BLK, BLK)], xin[j], si[j])

    def pair_body(m, carry):
        j0 = 2 * m
        for p in range(2):
            j = j0 + p
            off = base + j * BLK
            off_next = base + jnp.minimum(j + 2, NBLK - 2 + p) * BLK
            pltpu.make_async_copy(
                x_hbm.at[pl.ds(off, BLK)], xin[p], si[p]).wait()
            compute_block(xin[p], outb[p])
            pltpu.async_copy(outb[p], out_hbm.at[pl.ds(off, BLK)], so[p])
            pltpu.async_copy(x_hbm.at[pl.ds(off_next, BLK)], xin[p], si[p])
        for p in range(2):
            pltpu.make_async_copy(
                outb[p], out_hbm.at[pl.ds(base + (j0 + p) * BLK, BLK)],
                so[p]).wait()
        return carry

    lax.fori_loop(0, NBLK // 2, pair_body, 0)
    for p in range(2):
        pltpu.make_async_copy(
            x_hbm.at[pl.ds(base, BLK)], xin[p], si[p]).wait()


_spline_call = pl.kernel(
    _spline_body,
    out_type=jax.ShapeDtypeStruct((N_TOTAL,), jnp.float32),
    mesh=plsc.VectorSubcoreMesh(core_axis_name="c", subcore_axis_name="s"),
    compiler_params=pltpu.CompilerParams(needs_layout_passes=False),
    scratch_types=[
        pltpu.VMEM((NKNOTS,), jnp.float32),
        pltpu.VMEM((NKNOTS,), jnp.float32),
        pltpu.VMEM((NKNOTS,), jnp.float32),
        pltpu.VMEM((NKNOTS,), jnp.float32),
        pltpu.VMEM((NKNOTS,), jnp.float32),
        pltpu.VMEM((NKNOTS,), jnp.float32),
        pltpu.VMEM((NKNOTS,), jnp.int32),
        pltpu.VMEM((BLK,), jnp.float32),
        pltpu.VMEM((BLK,), jnp.float32),
        pltpu.VMEM((BLK,), jnp.float32),
        pltpu.VMEM((BLK,), jnp.float32),
        pltpu.SemaphoreType.DMA,
        pltpu.SemaphoreType.DMA,
        pltpu.SemaphoreType.DMA,
        pltpu.SemaphoreType.DMA,
    ],
)


def kernel(x, knots, coeffs):
    return _spline_call(x, knots, coeffs)


# speedup vs baseline: 1.5060x; 1.5060x over previous
import jax
import jax.numpy as jnp
from jax import lax
from jax.experimental import pallas as pl
from jax.experimental.pallas import tpu as pltpu
from jax.experimental.pallas import tpu_sc as plsc

NKNOTS = 4096
NINT = NKNOTS - 1
INV_H = NINT / 2.0
N_TOTAL = 8388608
NC, NS, L = 2, 16, 16
NW = NC * NS
CHUNK = N_TOTAL // NW
BLK = 16384
NBLK = CHUNK // BLK
VPB = BLK // L
MASK_HI = jnp.int32(-65536)
RND = jnp.int32(0x8000)


def _spline_body(x_hbm, knots_hbm, coeffs_hbm, out_hbm,
                 y_v, kn_v, h_v, dl_v, d_v, ab_v, ce_v,
                 xin0, xin1, outb0, outb1, si0, si1, so0, so1):
    wid = lax.axis_index("s") * NC + lax.axis_index("c")
    base = wid * CHUNK
    lanes = lax.iota(jnp.int32, L)

    pltpu.sync_copy(coeffs_hbm, y_v)
    pltpu.sync_copy(knots_hbm, kn_v)

    @plsc.parallel_loop(0, NKNOTS // L, unroll=4)
    def pass1(k):
        i0 = k * L
        idx = lanes + i0
        idxp = jnp.minimum(idx + 1, NKNOTS - 1)
        y0 = y_v[pl.ds(i0, L)]
        y1 = plsc.load_gather(y_v, [idxp])
        k0 = kn_v[pl.ds(i0, L)]
        k1 = plsc.load_gather(kn_v, [idxp])
        hh = (k1 - k0) + 1e-12
        h_v[pl.ds(i0, L)] = hh
        dl_v[pl.ds(i0, L)] = (y1 - y0) / hh

    @plsc.parallel_loop(0, NKNOTS // L, unroll=4)
    def pass2(k):
        i0 = k * L
        idx = lanes + i0
        idxm = jnp.maximum(idx - 1, 0)
        dln = dl_v[pl.ds(i0, L)]
        dlp = plsc.load_gather(dl_v, [idxm])
        hn = h_v[pl.ds(i0, L)]
        hp = plsc.load_gather(h_v, [idxm])
        same = (dlp * dln) > 0.0
        w1 = 2.0 * hn + hp
        w2 = hn + 2.0 * hp
        den = w1 / (dlp + 1e-12) + w2 / (dln + 1e-12) + 1e-12
        d = jnp.where(same, (w1 + w2) / den, 0.0)
        d = jnp.where(idx == 0, dln, d)
        d = jnp.where(idx == NKNOTS - 1, dlp, d)
        d_v[pl.ds(i0, L)] = d

    @plsc.parallel_loop(0, NKNOTS // L, unroll=4)
    def pass3(k):
        i0 = k * L
        idx = lanes + i0
        idxp = jnp.minimum(idx + 1, NKNOTS - 1)
        y0 = y_v[pl.ds(i0, L)]
        y1 = plsc.load_gather(y_v, [idxp])
        d0 = d_v[pl.ds(i0, L)]
        d1 = plsc.load_gather(d_v, [idxp])
        hh = h_v[pl.ds(i0, L)]
        b = hh * d0
        hd1 = hh * d1
        dy = y1 - y0
        c = 3.0 * dy - 2.0 * b - hd1
        e = -2.0 * dy + b + hd1
        abits = plsc.bitcast(y0, jnp.int32)
        bbits = plsc.bitcast(b, jnp.int32)
        cbits = plsc.bitcast(c, jnp.int32)
        ebits = plsc.bitcast(e, jnp.int32)
        wab = ((abits + RND) & MASK_HI) | lax.shift_right_logical(
            bbits + RND, 16)
        wce = ((cbits + RND) & MASK_HI) | lax.shift_right_logical(
            ebits + RND, 16)
        ab_v[pl.ds(i0, L)] = wab
        ce_v[pl.ds(i0, L)] = wce

    c4095 = jnp.full((L,), NKNOTS - 1, jnp.int32)
    c4094 = jnp.full((L,), NKNOTS - 2, jnp.int32)
    yl = plsc.load_gather(y_v, [c4095])
    shi = plsc.load_gather(h_v, [c4094]) * plsc.load_gather(d_v, [c4095])

    def compute_block(src_ref, dst_ref):
        @plsc.parallel_loop(0, VPB, unroll=8)
        def inner(i):
            xo = i * L
            xv = src_ref[pl.ds(xo, L)]
            ff = xv * INV_H + INV_H
            ffc = jnp.minimum(jnp.maximum(ff, 0.0), float(NINT - 1))
            fi = ffc.astype(jnp.int32)
            t = ff - fi.astype(jnp.float32)
            wab = plsc.load_gather(ab_v, [fi])
            wce = plsc.load_gather(ce_v, [fi])
            a = plsc.bitcast(wab & MASK_HI, jnp.float32)
            b = plsc.bitcast(lax.shift_left(wab, 16), jnp.float32)
            c = plsc.bitcast(wce & MASK_HI, jnp.float32)
            e = plsc.bitcast(lax.shift_left(wce, 16), jnp.float32)
            tcl = jnp.minimum(jnp.maximum(t, 0.0), 1.0)
            dt = t - tcl
            cub = a + tcl * (b + tcl * (c + tcl * e))
            bse = jnp.where(dt > 0.0, yl, cub)
            slp = jnp.where(dt < 0.0, b, shi)
            dst_ref[pl.ds(xo, L)] = bse + slp * dt

    xin = (xin0, xin1)
    outb = (outb0, outb1)
    si = (si0, si1)
    so = (so0, so1)

    for j in range(2):
        pltpu.async_copy(x_hbm.at[pl.ds(base + j * BLK, BLK)], xin[j], si[j])

    def pair_body(m, carry):
        j0 = 2 * m
        for p in range(2):
            j = j0 + p
            off = base + j * BLK
            off_next = base + jnp.minimum(j + 2, NBLK - 2 + p) * BLK
            pltpu.make_async_copy(
                x_hbm.at[pl.ds(off, BLK)], xin[p], si[p]).wait()
            compute_block(xin[p], outb[p])
            pltpu.async_copy(outb[p], out_hbm.at[pl.ds(off, BLK)], so[p])
            pltpu.async_copy(x_hbm.at[pl.ds(off_next, BLK)], xin[p], si[p])
        for p in range(2):
            pltpu.make_async_copy(
                outb[p], out_hbm.at[pl.ds(base + (j0 + p) * BLK, BLK)],
                so[p]).wait()
        return carry

    lax.fori_loop(0, NBLK // 2, pair_body, 0)
    for p in range(2):
        pltpu.make_async_copy(
            x_hbm.at[pl.ds(base, BLK)], xin[p], si[p]).wait()


_spline_call = pl.kernel(
    _spline_body,
    out_type=jax.ShapeDtypeStruct((N_TOTAL,), jnp.float32),
    mesh=plsc.VectorSubcoreMesh(core_axis_name="c", subcore_axis_name="s"),
    compiler_params=pltpu.CompilerParams(needs_layout_passes=False),
    scratch_types=[
        pltpu.VMEM((NKNOTS,), jnp.float32),
        pltpu.VMEM((NKNOTS,), jnp.float32),
        pltpu.VMEM((NKNOTS,), jnp.float32),
        pltpu.VMEM((NKNOTS,), jnp.float32),
        pltpu.VMEM((NKNOTS,), jnp.float32),
        pltpu.VMEM((NKNOTS,), jnp.int32),
        pltpu.VMEM((NKNOTS,), jnp.int32),
        pltpu.VMEM((BLK,), jnp.float32),
        pltpu.VMEM((BLK,), jnp.float32),
        pltpu.VMEM((BLK,), jnp.float32),
        pltpu.VMEM((BLK,), jnp.float32),
        pltpu.SemaphoreType.DMA,
        pltpu.SemaphoreType.DMA,
        pltpu.SemaphoreType.DMA,
        pltpu.SemaphoreType.DMA,
    ],
)


def kernel(x, knots, coeffs):
    return _spline_call(x, knots, coeffs)
